# baseline (speedup 1.0000x reference)
import functools

import jax
import jax.numpy as jnp
from jax.experimental import pallas as pl
from jax.experimental.pallas import tpu as pltpu


_B, _N, _NCLS = 8, 4096, 13


def _sqdist(src, dst):
    return (jnp.sum(src ** 2, -1)[:, :, None] + jnp.sum(dst ** 2, -1)[:, None, :]
            - 2.0 * jnp.matmul(src, dst.transpose(0, 2, 1)))


def _idx_pts(points, idx):
    b = points.shape[0]
    batch = jnp.arange(b).reshape((b,) + (1,) * (idx.ndim - 1))
    return points[batch, idx]


def _fps(xyz, npoint):
    b, n, _ = xyz.shape
    def body(i, state):
        centroids, distance, farthest = state
        centroids = centroids.at[:, i].set(farthest)
        centroid = _idx_pts(xyz, farthest[:, None])
        dist = jnp.sum((xyz - centroid) ** 2, -1)
        distance = jnp.minimum(distance, dist)
        farthest = jnp.argmax(distance, -1).astype(jnp.int32)
        return centroids, distance, farthest
    init = (jnp.zeros((b, npoint), jnp.int32), jnp.full((b, n), 1e10, jnp.float32),
            jnp.zeros((b,), jnp.int32))
    centroids, _, _ = jax.lax.fori_loop(0, npoint, body, init)
    return centroids


def _ball(radius, nsample, xyz, new_xyz):
    b, n, _ = xyz.shape
    s = new_xyz.shape[1]
    sqrdists = _sqdist(new_xyz, xyz)
    group_idx = jnp.broadcast_to(jnp.arange(n, dtype=jnp.int32), (b, s, n))
    group_idx = jnp.where(sqrdists > radius ** 2, n, group_idx)
    group_idx = jnp.sort(group_idx, axis=-1)[:, :, :nsample]
    group_first = group_idx[:, :, :1]
    group_idx = jnp.where(group_idx == n, group_first, group_idx)
    return group_idx


def _convbn2(p, x):
    x = jnp.einsum('oc,bcks->boks', p['W'], x) + p['b'][None, :, None, None]
    mean = jnp.mean(x, axis=(0, 2, 3), keepdims=True)
    var = jnp.var(x, axis=(0, 2, 3), keepdims=True)
    x = (x - mean) / jnp.sqrt(var + 1e-5)
    x = x * p['gamma'][None, :, None, None] + p['beta'][None, :, None, None]
    return jax.nn.relu(x)


def _convbn1(p, x):
    x = jnp.einsum('oc,bcn->bon', p['W'], x) + p['b'][None, :, None]
    mean = jnp.mean(x, axis=(0, 2), keepdims=True)
    var = jnp.var(x, axis=(0, 2), keepdims=True)
    x = (x - mean) / jnp.sqrt(var + 1e-5)
    x = x * p['gamma'][None, :, None] + p['beta'][None, :, None]
    return jax.nn.relu(x)


def _sa(ps, xyz, points, npoint, radius, nsample, group_all):
    xyz_t = xyz.transpose(0, 2, 1)
    points_t = points.transpose(0, 2, 1)
    if group_all:
        new_xyz = jnp.zeros((xyz_t.shape[0], 1, 3), jnp.float32)
        new_points = jnp.concatenate([xyz_t[:, None], points_t[:, None]], -1)
    else:
        fps_idx = _fps(xyz_t, npoint)
        new_xyz = _idx_pts(xyz_t, fps_idx)
        idx = _ball(radius, nsample, xyz_t, new_xyz)
        grouped_xyz = _idx_pts(xyz_t, idx) - new_xyz[:, :, None, :]
        grouped_points = _idx_pts(points_t, idx)
        new_points = jnp.concatenate([grouped_xyz, grouped_points], -1)
    x = new_points.transpose(0, 3, 2, 1)
    for p in ps:
        x = _convbn2(p, x)
    x = jnp.max(x, axis=2)
    return new_xyz.transpose(0, 2, 1), x


def _fp(ps, xyz1, xyz2, points1, points2):
    xyz1_t = xyz1.transpose(0, 2, 1)
    xyz2_t = xyz2.transpose(0, 2, 1)
    points2_t = points2.transpose(0, 2, 1)
    n = xyz1_t.shape[1]
    s = xyz2_t.shape[1]
    if s == 1:
        interpolated = jnp.repeat(points2_t, n, axis=1)
    else:
        dists = _sqdist(xyz1_t, xyz2_t)
        idx = jnp.argsort(dists, axis=-1)[:, :, :3]
        d3 = jnp.take_along_axis(dists, idx, axis=-1)
        dist_recip = 1.0 / (d3 + 1e-8)
        norm = jnp.sum(dist_recip, axis=2, keepdims=True)
        weight = dist_recip / norm
        interpolated = jnp.sum(_idx_pts(points2_t, idx) * weight[..., None], axis=2)
    points1_t = points1.transpose(0, 2, 1)
    x = jnp.concatenate([points1_t, interpolated], -1).transpose(0, 2, 1)
    for p in ps:
        x = _convbn1(p, x)
    return x


_CN = 512


def _head_stats_body(x_ref, w1_ref, b1_ref, feat_ref, sum_ref, ssq_ref):
    i = pl.program_id(0)
    j = pl.program_id(1)
    x = x_ref[0]
    feat = jnp.dot(w1_ref[...], x, preferred_element_type=jnp.float32)
    feat = feat + b1_ref[...].reshape(128, 1)
    feat_ref[0] = feat
    s = jnp.sum(feat, axis=1).reshape(1, 128)
    q = jnp.sum(feat * feat, axis=1).reshape(1, 128)

    @pl.when(jnp.logical_and(i == 0, j == 0))
    def _init():
        sum_ref[...] = jnp.zeros_like(sum_ref)
        ssq_ref[...] = jnp.zeros_like(ssq_ref)

    sum_ref[...] += s
    ssq_ref[...] += q


def _head_norm_body(feat_ref, sum_ref, ssq_ref, g1_ref, be1_ref, w2_ref,
                    b2_ref, out_ref):
    cnt = float(_B * _N)
    mean = sum_ref[...] / cnt
    var = ssq_ref[...] / cnt - mean * mean
    scale = g1_ref[...].reshape(1, 128) / jnp.sqrt(var + 1e-5)
    shift = be1_ref[...].reshape(1, 128) - mean * scale
    feat = feat_ref[0]
    feat = feat * scale.reshape(128, 1) + shift.reshape(128, 1)
    feat = jnp.maximum(feat, 0.0)
    logits = jnp.dot(w2_ref[...], feat, preferred_element_type=jnp.float32)
    logits = logits + b2_ref[...].reshape(_NCLS, 1)
    m = jnp.max(logits, axis=0, keepdims=True)
    z = logits - m
    lse = jnp.log(jnp.sum(jnp.exp(z), axis=0, keepdims=True))
    out_ref[0] = (z - lse).T


def _head(h, x):
    grid = (_B, _N // _CN)
    feat, ssum, ssq = pl.pallas_call(
        _head_stats_body,
        grid=grid,
        in_specs=[
            pl.BlockSpec((1, 128, _CN), lambda i, j: (i, 0, j)),
            pl.BlockSpec((128, 128), lambda i, j: (0, 0)),
            pl.BlockSpec((128,), lambda i, j: (0,)),
        ],
        out_specs=[
            pl.BlockSpec((1, 128, _CN), lambda i, j: (i, 0, j)),
            pl.BlockSpec((1, 128), lambda i, j: (0, 0)),
            pl.BlockSpec((1, 128), lambda i, j: (0, 0)),
        ],
        out_shape=[
            jax.ShapeDtypeStruct((_B, 128, _N), jnp.float32),
            jax.ShapeDtypeStruct((1, 128), jnp.float32),
            jax.ShapeDtypeStruct((1, 128), jnp.float32),
        ],
    )(x, h['W1'], h['b1'])

    out = pl.pallas_call(
        _head_norm_body,
        grid=grid,
        in_specs=[
            pl.BlockSpec((1, 128, _CN), lambda i, j: (i, 0, j)),
            pl.BlockSpec((1, 128), lambda i, j: (0, 0)),
            pl.BlockSpec((1, 128), lambda i, j: (0, 0)),
            pl.BlockSpec((128,), lambda i, j: (0,)),
            pl.BlockSpec((128,), lambda i, j: (0,)),
            pl.BlockSpec((_NCLS, 128), lambda i, j: (0, 0)),
            pl.BlockSpec((_NCLS,), lambda i, j: (0,)),
        ],
        out_specs=pl.BlockSpec((1, _CN, _NCLS), lambda i, j: (i, j, 0)),
        out_shape=jax.ShapeDtypeStruct((_B, _N, _NCLS), jnp.float32),
    )(feat, ssum, ssq, h['gamma1'], h['beta1'], h['W2'], h['b2'])
    return out


def kernel(xyz, params):
    x = xyz.transpose(0, 2, 1)
    l0_xyz, l0_points = x, x
    l1_xyz, l1_points = _sa(params['sa1'], l0_xyz, l0_points, 512, 0.2, 32, False)
    l2_xyz, l2_points = _sa(params['sa2'], l1_xyz, l1_points, 128, 0.4, 64, False)
    l3_xyz, l3_points = _sa(params['sa3'], l2_xyz, l2_points, None, None, None, True)
    l2_points = _fp(params['fp3'], l2_xyz, l3_xyz, l2_points, l3_points)
    l1_points = _fp(params['fp2'], l1_xyz, l2_xyz, l1_points, l2_points)
    l0_points = _fp(params['fp1'], l0_xyz, l1_xyz, l0_points, l1_points)
    return _head(params['head'], l0_points)


# speedup vs baseline: 1.4223x; 1.4223x over previous
import functools

import jax
import jax.numpy as jnp
from jax.experimental import pallas as pl
from jax.experimental.pallas import tpu as pltpu


_B, _N, _NCLS = 8, 4096, 13


def _sqdist(src, dst):
    return (jnp.sum(src ** 2, -1)[:, :, None] + jnp.sum(dst ** 2, -1)[:, None, :]
            - 2.0 * jnp.matmul(src, dst.transpose(0, 2, 1)))


def _idx_pts(points, idx):
    b = points.shape[0]
    batch = jnp.arange(b).reshape((b,) + (1,) * (idx.ndim - 1))
    return points[batch, idx]


def _fps_body(s, x_ref, y_ref, z_ref, jf_ref, idx_ref):
    b, n = x_ref.shape
    x = x_ref[...]
    y = y_ref[...]
    z = z_ref[...]
    jf = jf_ref[...]
    iota = jax.lax.broadcasted_iota(jnp.int32, (b, n), 1)
    slot = jax.lax.broadcasted_iota(jnp.int32, (b, s), 1)

    def body(i, carry):
        dist, far, oi = carry
        sel = iota == far
        cx = jnp.sum(jnp.where(sel, x, 0.0), axis=1, keepdims=True)
        cy = jnp.sum(jnp.where(sel, y, 0.0), axis=1, keepdims=True)
        cz = jnp.sum(jnp.where(sel, z, 0.0), axis=1, keepdims=True)
        far_f = jnp.sum(jnp.where(sel, jf, 0.0), axis=1, keepdims=True)
        oi = jnp.where(slot == i, far_f, oi)
        dx = x - cx
        dy = y - cy
        dz = z - cz
        d = (dx * dx + dy * dy) + dz * dz
        dist = jnp.minimum(dist, d)
        md = jnp.max(dist, axis=1, keepdims=True)
        far = jnp.min(jnp.where(dist == md, iota, n), axis=1, keepdims=True)
        return dist, far, oi

    _, _, oi = jax.lax.fori_loop(
        0, s, body,
        (jnp.full((b, n), 1e10, jnp.float32), jnp.zeros((b, 1), jnp.int32),
         jnp.zeros((b, s), jnp.float32)))
    idx_ref[...] = oi.astype(jnp.int32)


def _fps_xla(xyz, npoint):
    b, n, _ = xyz.shape
    def body(i, state):
        centroids, distance, farthest = state
        centroids = centroids.at[:, i].set(farthest)
        centroid = _idx_pts(xyz, farthest[:, None])
        dist = jnp.sum((xyz - centroid) ** 2, -1)
        distance = jnp.minimum(distance, dist)
        farthest = jnp.argmax(distance, -1).astype(jnp.int32)
        return centroids, distance, farthest
    init = (jnp.zeros((b, npoint), jnp.int32), jnp.full((b, n), 1e10, jnp.float32),
            jnp.zeros((b,), jnp.int32))
    centroids, _, _ = jax.lax.fori_loop(0, npoint, body, init)
    return centroids


def _fps_idx(x, y, z, npoint):
    b, n = x.shape
    jf = jnp.broadcast_to(jnp.arange(n, dtype=jnp.float32), (b, n))
    return pl.pallas_call(
        functools.partial(_fps_body, npoint),
        out_shape=jax.ShapeDtypeStruct((b, npoint), jnp.int32),
    )(x, y, z, jf)


def _ball(radius, nsample, xyz, new_xyz):
    b, n, _ = xyz.shape
    s = new_xyz.shape[1]
    sqrdists = _sqdist(new_xyz, xyz)
    group_idx = jnp.broadcast_to(jnp.arange(n, dtype=jnp.int32), (b, s, n))
    group_idx = jnp.where(sqrdists > radius ** 2, n, group_idx)
    group_idx = jnp.sort(group_idx, axis=-1)[:, :, :nsample]
    group_first = group_idx[:, :, :1]
    group_idx = jnp.where(group_idx == n, group_first, group_idx)
    return group_idx


def _convbn2(p, x):
    x = jnp.einsum('oc,bcks->boks', p['W'], x) + p['b'][None, :, None, None]
    mean = jnp.mean(x, axis=(0, 2, 3), keepdims=True)
    var = jnp.var(x, axis=(0, 2, 3), keepdims=True)
    x = (x - mean) / jnp.sqrt(var + 1e-5)
    x = x * p['gamma'][None, :, None, None] + p['beta'][None, :, None, None]
    return jax.nn.relu(x)


def _convbn1(p, x):
    x = jnp.einsum('oc,bcn->bon', p['W'], x) + p['b'][None, :, None]
    mean = jnp.mean(x, axis=(0, 2), keepdims=True)
    var = jnp.var(x, axis=(0, 2), keepdims=True)
    x = (x - mean) / jnp.sqrt(var + 1e-5)
    x = x * p['gamma'][None, :, None] + p['beta'][None, :, None]
    return jax.nn.relu(x)


def _sa(ps, xyz, points, new_xyz, radius, nsample, group_all):
    xyz_t = xyz.transpose(0, 2, 1)
    points_t = points.transpose(0, 2, 1)
    if group_all:
        new_xyz = jnp.zeros((xyz_t.shape[0], 1, 3), jnp.float32)
        new_points = jnp.concatenate([xyz_t[:, None], points_t[:, None]], -1)
    else:
        idx = _ball(radius, nsample, xyz_t, new_xyz)
        grouped_xyz = _idx_pts(xyz_t, idx) - new_xyz[:, :, None, :]
        grouped_points = _idx_pts(points_t, idx)
        new_points = jnp.concatenate([grouped_xyz, grouped_points], -1)
    x = new_points.transpose(0, 3, 2, 1)
    for p in ps:
        x = _convbn2(p, x)
    x = jnp.max(x, axis=2)
    return new_xyz.transpose(0, 2, 1), x


def _top3_body(a_ref, bt_ref, i_ref, d_ref):
    a = a_ref[0]
    bt = bt_ref[0]
    r = a.shape[0]
    s = bt.shape[1]
    af = a.astype(jnp.bfloat16).astype(jnp.float32)
    btf = bt.astype(jnp.bfloat16).astype(jnp.float32)
    m = ((af[:, 0:1] * btf[0:1, :] + af[:, 1:2] * btf[1:2, :])
         + af[:, 2:3] * btf[2:3, :])
    a2 = (a[:, 0:1] * a[:, 0:1] + a[:, 1:2] * a[:, 1:2]) + a[:, 2:3] * a[:, 2:3]
    b2 = (bt[0:1, :] * bt[0:1, :] + bt[1:2, :] * bt[1:2, :]) + bt[2:3, :] * bt[2:3, :]
    d = (a2 + b2) - 2.0 * m
    iota = jax.lax.broadcasted_iota(jnp.int32, (r, s), 1)
    remaining = d
    picks = []
    vals = []
    for _ in range(3):
        m3 = jnp.min(remaining, axis=1, keepdims=True)
        pick = jnp.min(jnp.where(remaining == m3, iota, s), axis=1, keepdims=True)
        picks.append(pick)
        vals.append(m3)
        remaining = jnp.where(iota == pick, 1e30, remaining)
    i_ref[0] = jnp.concatenate(picks, axis=1)
    d_ref[0] = jnp.concatenate(vals, axis=1)


def _top3(xyz1_t, xyz2_t):
    b, n, _ = xyz1_t.shape
    s = xyz2_t.shape[1]
    a = jnp.pad(xyz1_t, ((0, 0), (0, 0), (0, 5)))
    bt = jnp.pad(xyz2_t, ((0, 0), (0, 0), (0, 5))).transpose(0, 2, 1)
    rblk = 512
    return pl.pallas_call(
        _top3_body,
        grid=(b, n // rblk),
        in_specs=[pl.BlockSpec((1, rblk, 8), lambda i, j: (i, j, 0)),
                  pl.BlockSpec((1, 8, s), lambda i, j: (i, 0, 0))],
        out_specs=[pl.BlockSpec((1, rblk, 3), lambda i, j: (i, j, 0)),
                   pl.BlockSpec((1, rblk, 3), lambda i, j: (i, j, 0))],
        out_shape=[jax.ShapeDtypeStruct((b, n, 3), jnp.int32),
                   jax.ShapeDtypeStruct((b, n, 3), jnp.float32)],
    )(a, bt)


def _fp(ps, xyz1, xyz2, points1, points2):
    xyz1_t = xyz1.transpose(0, 2, 1)
    xyz2_t = xyz2.transpose(0, 2, 1)
    points2_t = points2.transpose(0, 2, 1)
    n = xyz1_t.shape[1]
    s = xyz2_t.shape[1]
    if s == 1:
        interpolated = jnp.repeat(points2_t, n, axis=1)
    else:
        dists = _sqdist(xyz1_t, xyz2_t)
        idx = jnp.argsort(dists, axis=-1)[:, :, :3]
        d3 = jnp.take_along_axis(dists, idx, axis=-1)
        dist_recip = 1.0 / (d3 + 1e-8)
        norm = jnp.sum(dist_recip, axis=2, keepdims=True)
        weight = dist_recip / norm
        interpolated = jnp.sum(_idx_pts(points2_t, idx) * weight[..., None], axis=2)
    points1_t = points1.transpose(0, 2, 1)
    x = jnp.concatenate([points1_t, interpolated], -1).transpose(0, 2, 1)
    for p in ps:
        x = _convbn1(p, x)
    return x


_CN = 512


def _head_stats_body(x_ref, w1_ref, b1_ref, feat_ref, sum_ref, ssq_ref):
    i = pl.program_id(0)
    j = pl.program_id(1)
    x = x_ref[0]
    feat = jnp.dot(w1_ref[...], x, preferred_element_type=jnp.float32)
    feat = feat + b1_ref[...].reshape(128, 1)
    feat_ref[0] = feat
    s = jnp.sum(feat, axis=1).reshape(1, 128)
    q = jnp.sum(feat * feat, axis=1).reshape(1, 128)

    @pl.when(jnp.logical_and(i == 0, j == 0))
    def _init():
        sum_ref[...] = jnp.zeros_like(sum_ref)
        ssq_ref[...] = jnp.zeros_like(ssq_ref)

    sum_ref[...] += s
    ssq_ref[...] += q


def _head_norm_body(feat_ref, sum_ref, ssq_ref, g1_ref, be1_ref, w2_ref,
                    b2_ref, out_ref):
    cnt = float(_B * _N)
    mean = sum_ref[...] / cnt
    var = ssq_ref[...] / cnt - mean * mean
    scale = g1_ref[...].reshape(1, 128) / jnp.sqrt(var + 1e-5)
    shift = be1_ref[...].reshape(1, 128) - mean * scale
    feat = feat_ref[0]
    feat = feat * scale.reshape(128, 1) + shift.reshape(128, 1)
    feat = jnp.maximum(feat, 0.0)
    logits = jnp.dot(w2_ref[...], feat, preferred_element_type=jnp.float32)
    logits = logits + b2_ref[...].reshape(_NCLS, 1)
    m = jnp.max(logits, axis=0, keepdims=True)
    z = logits - m
    lse = jnp.log(jnp.sum(jnp.exp(z), axis=0, keepdims=True))
    out_ref[0] = (z - lse).T


def _head(h, x):
    grid = (_B, _N // _CN)
    feat, ssum, ssq = pl.pallas_call(
        _head_stats_body,
        grid=grid,
        in_specs=[
            pl.BlockSpec((1, 128, _CN), lambda i, j: (i, 0, j)),
            pl.BlockSpec((128, 128), lambda i, j: (0, 0)),
            pl.BlockSpec((128,), lambda i, j: (0,)),
        ],
        out_specs=[
            pl.BlockSpec((1, 128, _CN), lambda i, j: (i, 0, j)),
            pl.BlockSpec((1, 128), lambda i, j: (0, 0)),
            pl.BlockSpec((1, 128), lambda i, j: (0, 0)),
        ],
        out_shape=[
            jax.ShapeDtypeStruct((_B, 128, _N), jnp.float32),
            jax.ShapeDtypeStruct((1, 128), jnp.float32),
            jax.ShapeDtypeStruct((1, 128), jnp.float32),
        ],
    )(x, h['W1'], h['b1'])

    out = pl.pallas_call(
        _head_norm_body,
        grid=grid,
        in_specs=[
            pl.BlockSpec((1, 128, _CN), lambda i, j: (i, 0, j)),
            pl.BlockSpec((1, 128), lambda i, j: (0, 0)),
            pl.BlockSpec((1, 128), lambda i, j: (0, 0)),
            pl.BlockSpec((128,), lambda i, j: (0,)),
            pl.BlockSpec((128,), lambda i, j: (0,)),
            pl.BlockSpec((_NCLS, 128), lambda i, j: (0, 0)),
            pl.BlockSpec((_NCLS,), lambda i, j: (0,)),
        ],
        out_specs=pl.BlockSpec((1, _CN, _NCLS), lambda i, j: (i, j, 0)),
        out_shape=jax.ShapeDtypeStruct((_B, _N, _NCLS), jnp.float32),
    )(feat, ssum, ssq, h['gamma1'], h['beta1'], h['W2'], h['b2'])
    return out


def kernel(xyz, params):
    x = xyz.transpose(0, 2, 1)
    l0_xyz, l0_points = x, x
    xyz_t = xyz
    fi1 = _fps_idx(xyz[:, :, 0], xyz[:, :, 1], xyz[:, :, 2], 512)
    nx1 = _idx_pts(xyz_t, fi1)
    l1_xyz, l1_points = _sa(params['sa1'], l0_xyz, l0_points, nx1, 0.2, 32, False)
    fi2 = _fps_xla(nx1, 128)
    nx2 = _idx_pts(nx1, fi2)
    l2_xyz, l2_points = _sa(params['sa2'], l1_xyz, l1_points, nx2, 0.4, 64, False)
    l3_xyz, l3_points = _sa(params['sa3'], l2_xyz, l2_points, None, None, None, True)
    l2_points = _fp(params['fp3'], l2_xyz, l3_xyz, l2_points, l3_points)
    l1_points = _fp(params['fp2'], l1_xyz, l2_xyz, l1_points, l2_points)
    l0_points = _fp(params['fp1'], l0_xyz, l1_xyz, l0_points, l1_points)
    return _head(params['head'], l0_points)
